# Initial kernel scaffold; baseline (speedup 1.0000x reference)
import jax
import jax.numpy as jnp
from jax.experimental import pallas as pl


def kernel(x, edge_index, W1_l, W1_r, b1, W2_l, W2_r, b2):
    raise NotImplementedError("write your pallas kernel here")


# speedup vs baseline: 8.4513x; 8.4513x over previous
import jax
import jax.numpy as jnp
from jax import lax
from jax.experimental import pallas as pl
from jax.experimental.pallas import tpu as pltpu
from jax.experimental.pallas import tpu_sc as plsc

NC = 2
NS = 16
LANES = 16
NW = NC * NS

LDEG = 16
ZB = 200


def _pick_chunk(edges_per_worker):
    for k in range(128, 7, -8):
        if edges_per_worker % k == 0:
            return k
    raise ValueError(f"no valid chunk size for {edges_per_worker}")


def _make_sc_aggregate(n, d, e, with_deg):
    dh = d // NC
    ew = e // NS
    k = _pick_chunk(ew)
    nch = ew // k
    nb = n // ZB
    tmax = -(-nb // NS)

    mesh = plsc.VectorSubcoreMesh(core_axis_name="c", subcore_axis_name="s")

    out_type = [jax.ShapeDtypeStruct((NC, n, dh), jnp.float32)]
    scratch = [
        pltpu.VMEM((nch, k), jnp.int32),
        pltpu.VMEM((nch, k), jnp.int32),
        pltpu.VMEM((2, k, dh), jnp.float32),
        pltpu.VMEM((ZB, dh), jnp.float32),
        pltpu.SemaphoreType.DMA,
        pltpu.SemaphoreType.DMA,
        pltpu.VMEM_SHARED((n, dh), jnp.float32),
    ]
    if with_deg:
        out_type.append(jax.ShapeDtypeStruct((NC, n, LDEG), jnp.float32))
        scratch += [
            pltpu.VMEM((k, LDEG), jnp.float32),
            pltpu.VMEM((ZB, LDEG), jnp.float32),
            pltpu.VMEM_SHARED((n, LDEG), jnp.float32),
        ]

    def body(table, srcw, dstw, *refs):
        if with_deg:
            (out, dego, srcv, dstv, bufs, zbuf, sem0, sem1, acc,
             ones, zdeg, dacc) = refs
        else:
            out, srcv, dstv, bufs, zbuf, sem0, sem1, acc = refs
        sems = (sem0, sem1)
        c = lax.axis_index("c")
        s = lax.axis_index("s")
        wid = c * NS + s
        z16 = jnp.zeros((LANES,), jnp.float32)

        @pl.loop(0, ZB)
        def _(r):
            @pl.loop(0, dh, step=LANES)
            def _(cc):
                zbuf[r, pl.ds(cc, LANES)] = z16

        @pl.loop(0, tmax)
        def _(t):
            bid = s + t * NS

            @pl.when(bid < nb)
            def _():
                pltpu.sync_copy(zbuf, acc.at[pl.ds(bid * ZB, ZB)])

        if with_deg:
            o16 = jnp.ones((LANES,), jnp.float32)

            @pl.loop(0, ZB)
            def _(r):
                zdeg[r, pl.ds(0, LDEG)] = z16

            @pl.loop(0, k)
            def _(r):
                ones[r, pl.ds(0, LDEG)] = o16

            @pl.loop(0, tmax)
            def _(t):
                bid = s + t * NS

                @pl.when(bid < nb)
                def _():
                    pltpu.sync_copy(zdeg, dacc.at[pl.ds(bid * ZB, ZB)])

        pltpu.sync_copy(srcw.at[wid], srcv)
        pltpu.sync_copy(dstw.at[s], dstv)

        plsc.subcore_barrier()

        count_deg = with_deg

        def start(j, b):
            pltpu.async_copy(table.at[srcv.at[j]], bufs.at[b], sems[b])

        def finish(j, b):
            pltpu.make_async_copy(table.at[srcv.at[j]], bufs.at[b],
                                  sems[b]).wait()
            pltpu.sync_copy(bufs.at[b], acc.at[dstv.at[j]], add=True)
            if count_deg:
                @pl.when(c == 0)
                def _():
                    pltpu.sync_copy(ones, dacc.at[dstv.at[j]], add=True)

        start(0, 0)

        @pl.loop(0, nch, step=2)
        def _(j):
            @pl.when(j + 1 < nch)
            def _():
                start(j + 1, 1)

            finish(j, 0)

            @pl.when(j + 2 < nch)
            def _():
                start(j + 2, 0)

            @pl.when(j + 1 < nch)
            def _():
                finish(j + 1, 1)

        plsc.subcore_barrier()

        @pl.loop(0, tmax)
        def _(t):
            bid = s + t * NS

            @pl.when(bid < nb)
            def _():
                pltpu.sync_copy(acc.at[pl.ds(bid * ZB, ZB)],
                                out.at[c, pl.ds(bid * ZB, ZB)])

        if with_deg:
            @pl.loop(0, tmax)
            def _(t):
                bid = s + t * NS

                @pl.when(bid < nb)
                def _():
                    pltpu.sync_copy(dacc.at[pl.ds(bid * ZB, ZB)],
                                    dego.at[c, pl.ds(bid * ZB, ZB)])

    return pl.kernel(
        body, out_type=out_type, mesh=mesh, scratch_types=scratch,
        compiler_params=pltpu.CompilerParams(use_tc_tiling_on_sc=False))


RB = 2000


def _mm_in(x, wcat, bcat, d):
    n = x.shape[0]
    d_in = x.shape[1]
    dh = d // NC

    def body(x_ref, w_ref, b_ref, o1_ref, o2_ref):
        h = jnp.dot(x_ref[...], w_ref[...],
                    preferred_element_type=jnp.float32,
                    precision=lax.Precision.HIGHEST) + b_ref[...]
        o1_ref[0] = h[:, :dh]
        o1_ref[1] = h[:, dh:d]
        o2_ref[...] = h[:, d:]

    return pl.pallas_call(
        body,
        grid=(n // RB,),
        in_specs=[pl.BlockSpec((RB, d_in), lambda i: (i, 0)),
                  pl.BlockSpec(wcat.shape, lambda i: (0, 0)),
                  pl.BlockSpec(bcat.shape, lambda i: (0, 0))],
        out_specs=[pl.BlockSpec((NC, RB, dh), lambda i: (0, i, 0)),
                   pl.BlockSpec((RB, d), lambda i: (i, 0))],
        out_shape=[jax.ShapeDtypeStruct((NC, n, dh), jnp.float32),
                   jax.ShapeDtypeStruct((n, d), jnp.float32)],
    )(x, wcat, bcat)


def _mid(aggp, degp, xr, wcat, bcat, d):
    n, d_in = xr.shape
    dha = aggp.shape[2]
    dh = d // NC

    def body(a_ref, g_ref, xr_ref, w_ref, b_ref, o1_ref, o2_ref):
        agg = jnp.concatenate([a_ref[0], a_ref[1]], axis=1)
        deg = (jnp.max(g_ref[0], axis=1, keepdims=True)
               + jnp.max(g_ref[1], axis=1, keepdims=True))
        dinv = 1.0 / jnp.maximum(deg, 1.0)
        h = jnp.maximum(agg * dinv + xr_ref[...], 0.0)
        hcat = jnp.dot(h, w_ref[...],
                       preferred_element_type=jnp.float32,
                       precision=lax.Precision.HIGHEST) + b_ref[...]
        o1_ref[0] = hcat[:, :dh]
        o1_ref[1] = hcat[:, dh:d]
        o2_ref[...] = hcat[:, d:]

    return pl.pallas_call(
        body,
        grid=(n // RB,),
        in_specs=[pl.BlockSpec((NC, RB, dha), lambda i: (0, i, 0)),
                  pl.BlockSpec((NC, RB, LDEG), lambda i: (0, i, 0)),
                  pl.BlockSpec((RB, d_in), lambda i: (i, 0)),
                  pl.BlockSpec(wcat.shape, lambda i: (0, 0)),
                  pl.BlockSpec(bcat.shape, lambda i: (0, 0))],
        out_specs=[pl.BlockSpec((NC, RB, dh), lambda i: (0, i, 0)),
                   pl.BlockSpec((RB, d), lambda i: (i, 0))],
        out_shape=[jax.ShapeDtypeStruct((NC, n, dh), jnp.float32),
                   jax.ShapeDtypeStruct((n, d), jnp.float32)],
    )(aggp, degp, xr, wcat, bcat)


def _epilogue(aggp, degp, hr):
    n, d = hr.shape
    dha = aggp.shape[2]

    def body(a_ref, g_ref, hr_ref, o_ref):
        agg = jnp.concatenate([a_ref[0], a_ref[1]], axis=1)
        deg = (jnp.max(g_ref[0], axis=1, keepdims=True)
               + jnp.max(g_ref[1], axis=1, keepdims=True))
        dinv = 1.0 / jnp.maximum(deg, 1.0)
        o_ref[...] = agg * dinv + hr_ref[...]

    return pl.pallas_call(
        body,
        grid=(n // RB,),
        in_specs=[pl.BlockSpec((NC, RB, dha), lambda i: (0, i, 0)),
                  pl.BlockSpec((NC, RB, LDEG), lambda i: (0, i, 0)),
                  pl.BlockSpec((RB, d), lambda i: (i, 0))],
        out_specs=pl.BlockSpec((RB, d), lambda i: (i, 0)),
        out_shape=jax.ShapeDtypeStruct(hr.shape, jnp.float32),
    )(aggp, degp, hr)


def kernel(x, edge_index, W1_l, W1_r, b1, W2_l, W2_r, b2):
    n, d_in = x.shape
    e = edge_index.shape[1]
    d_hid = W1_l.shape[0]
    d_out = W2_l.shape[0]

    ei = edge_index.astype(jnp.int32)
    ew = e // NS
    k = _pick_chunk(ew)
    nch = ew // k
    src2 = ei[0].reshape(NS, nch, k)
    srcw = jnp.concatenate([src2, src2 + n], axis=0)
    dstw = ei[1].reshape(NS, nch, k)

    w1cat = jnp.concatenate([W1_l.T, W1_r.T], axis=1)
    b1cat = jnp.concatenate([jnp.zeros_like(b1), b1]).reshape(1, 2 * d_hid)
    w2cat = jnp.concatenate([W2_l.T, W2_r.T], axis=1)
    b2cat = jnp.concatenate([jnp.zeros_like(b2), b2]).reshape(1, 2 * d_out)

    agg1 = _make_sc_aggregate(n, d_hid, e, with_deg=True)
    agg2 = _make_sc_aggregate(n, d_out, e, with_deg=False)

    xl, xr = _mm_in(x, w1cat, b1cat, d_hid)
    aggp1, degp = agg1(xl.reshape(NC * n, d_hid // NC), srcw, dstw)
    hl, hr = _mid(aggp1, degp, xr, w2cat, b2cat, d_out)
    aggp2 = agg2(hl.reshape(NC * n, d_out // NC), srcw, dstw)
    if isinstance(aggp2, (list, tuple)):
        aggp2 = aggp2[0]
    return _epilogue(aggp2, degp, hr)
